# baseline (speedup 1.0000x reference)
import functools

import jax
import jax.numpy as jnp
from jax import lax
from jax.experimental import pallas as pl
from jax.experimental.pallas import tpu as pltpu
from jax.experimental.pallas import tpu_sc as plsc

_NUM_CORES = 2
_NUM_SUBCORES = 16
_NUM_WORKERS = _NUM_CORES * _NUM_SUBCORES
_L = 16
_CHUNK = 32
_NBUF = 3
_LAG = 1


def kernel(data, partitions):
    n_rows, n_cols = data.shape
    partitions = partitions.astype(jnp.int32)
    rows_per_w = n_rows // _NUM_WORKERS
    n_vecs_w = rows_per_w // _L
    n_chunks = rows_per_w // _CHUNK
    vecs_per_chunk = _CHUNK // _L
    mesh = plsc.VectorSubcoreMesh(
        core_axis_name="c", subcore_axis_name="s",
        num_cores=_NUM_CORES, num_subcores=_NUM_SUBCORES)

    @functools.partial(
        pl.kernel,
        mesh=mesh,
        compiler_params=pltpu.CompilerParams(needs_layout_passes=False),
        out_type=jax.ShapeDtypeStruct((n_rows, n_cols), data.dtype),
        scratch_types=[
            pltpu.VMEM((2, rows_per_w), jnp.int32),
            pltpu.VMEM((rows_per_w,), jnp.int32),
            pltpu.VMEM((n_chunks, _CHUNK), jnp.int32),
            pltpu.VMEM((_NBUF, _CHUNK, n_cols), jnp.float32),
            pltpu.SemaphoreType.DMA,
            pltpu.SemaphoreType.DMA,
            pltpu.SemaphoreType.DMA,
            pltpu.SemaphoreType.DMA,
            pltpu.SemaphoreType.DMA,
            pltpu.SemaphoreType.DMA,
            pltpu.SemaphoreType.DMA,
            pltpu.SemaphoreType.DMA,
            pltpu.SemaphoreType.DMA,
        ],
    )
    def run(data_hbm, part_hbm, out_hbm, pstage, pown, idx, buf, *sems):
        spart = sems[:2]
        sown = sems[2]
        sin = sems[3:3 + _NBUF]
        sout = sems[3 + _NBUF:3 + 2 * _NBUF]
        wid = lax.axis_index("s") * _NUM_CORES + lax.axis_index("c")
        base = wid * rows_per_w
        zero = jnp.zeros((_L,), jnp.int32)

        pltpu.async_copy(part_hbm.at[pl.ds(0, rows_per_w)],
                         pstage.at[0], spart[0])
        pltpu.async_copy(part_hbm.at[pl.ds(base, rows_per_w)], pown, sown)

        def scan_pair(t2, carry):
            tot, pre = carry
            for parity in range(2):
                t = t2 * 2 + parity

                @pl.when(t + 1 < _NUM_WORKERS)
                def _():
                    off = (t + 1) * rows_per_w
                    pltpu.async_copy(part_hbm.at[pl.ds(off, rows_per_w)],
                                     pstage.at[1 - parity],
                                     spart[1 - parity])

                pltpu.make_async_copy(
                    part_hbm.at[pl.ds(t * rows_per_w, rows_per_w)],
                    pstage.at[parity], spart[parity]).wait()
                piece = zero
                for j in range(n_vecs_w):
                    piece = piece + pstage[parity, pl.ds(j * _L, _L)]
                tot = tot + piece
                pre = pre + jnp.where(jnp.full((_L,), t < wid, jnp.bool_),
                                      piece, zero)
            return tot, pre

        tot_v, pre_v = lax.fori_loop(0, _NUM_WORKERS // 2, scan_pair,
                                     (zero, zero))
        ones_before = jnp.sum(pre_v)
        ones_total = jnp.sum(tot_v)
        zeros_total = n_rows - ones_total

        iota = lax.iota(jnp.int32, _L)
        pltpu.make_async_copy(part_hbm.at[pl.ds(base, rows_per_w)],
                              pown, sown).wait()

        def dest_body(k, ones_run):
            for j in range(vecs_per_chunk):
                kv = k * vecs_per_chunk + j
                v = pown[pl.ds(kv * _L, _L)]
                incl = plsc.cumsum(v)
                ones_excl = ones_run + incl - v
                row = base + kv * _L + iota
                dest = jnp.where(v == 0, row - ones_excl,
                                 zeros_total + ones_excl)
                idx[k, pl.ds(j * _L, _L)] = dest
                ones_run = ones_run + jnp.max(incl)
            return ones_run

        lax.fori_loop(0, n_chunks, dest_body, ones_before)

        def start_in(k):
            pltpu.async_copy(data_hbm.at[idx.at[k]],
                             buf.at[k % _NBUF], sin[k % _NBUF])

        def wait_in(k):
            pltpu.make_async_copy(data_hbm.at[idx.at[k]],
                                  buf.at[k % _NBUF], sin[k % _NBUF]).wait()

        def start_out(k):
            pltpu.async_copy(buf.at[k % _NBUF],
                             out_hbm.at[idx.at[k]], sout[k % _NBUF])

        def wait_out(k):
            pltpu.make_async_copy(buf.at[k % _NBUF],
                                  out_hbm.at[idx.at[k]],
                                  sout[k % _NBUF]).wait()

        for k in range(n_chunks + _LAG):
            if k < n_chunks:
                if k >= _NBUF:
                    wait_out(k - _NBUF)
                start_in(k)
            j = k - _LAG
            if 0 <= j < n_chunks:
                wait_in(j)
                start_out(j)
        for j in range(n_chunks - _NBUF, n_chunks):
            wait_out(j)

    return run(data, partitions)


# speedup vs baseline: 1.0983x; 1.0983x over previous
import functools

import jax
import jax.numpy as jnp
from jax import lax
from jax.experimental import pallas as pl
from jax.experimental.pallas import tpu as pltpu
from jax.experimental.pallas import tpu_sc as plsc

_NUM_CORES = 2
_NUM_SUBCORES = 16
_NUM_WORKERS = _NUM_CORES * _NUM_SUBCORES
_L = 16
_CHUNK = 32
_NBUF = 2
_LAG = 1
_UNROLL = 8
_INDIRECT = False


def kernel(data, partitions):
    n_rows, n_cols = data.shape
    partitions = partitions.astype(jnp.int32)
    rows_per_w = n_rows // _NUM_WORKERS
    n_vecs_total = n_rows // _L
    n_vecs_w = rows_per_w // _L
    n_chunks = rows_per_w // _CHUNK
    vecs_per_chunk = _CHUNK // _L
    mesh = plsc.VectorSubcoreMesh(
        core_axis_name="c", subcore_axis_name="s",
        num_cores=_NUM_CORES, num_subcores=_NUM_SUBCORES)

    @functools.partial(
        pl.kernel,
        mesh=mesh,
        compiler_params=pltpu.CompilerParams(needs_layout_passes=False),
        out_type=jax.ShapeDtypeStruct((n_rows, n_cols), data.dtype),
        scratch_types=[
            pltpu.VMEM((n_rows,), jnp.int32),
            pltpu.VMEM((n_chunks, _CHUNK), jnp.int32),
            pltpu.VMEM((_NBUF, _CHUNK, n_cols), jnp.float32),
            pltpu.SemaphoreType.DMA,
            pltpu.SemaphoreType.DMA,
            pltpu.SemaphoreType.DMA,
            pltpu.SemaphoreType.DMA,
        ],
    )
    def run(data_hbm, part_hbm, out_hbm, pv, idx, buf, *sems):
        sin = sems[:_NBUF]
        sout = sems[_NBUF:]
        wid = lax.axis_index("s") * _NUM_CORES + lax.axis_index("c")
        base = wid * rows_per_w
        base_vec = wid * n_vecs_w
        zero = jnp.zeros((_L,), jnp.int32)

        pltpu.sync_copy(part_hbm, pv)

        def count_body(g, accs):
            new = []
            for j, a in enumerate(accs):
                off = (g * _UNROLL + 2 * j) * _L
                a = a + pv[pl.ds(off, _L)] + pv[pl.ds(off + _L, _L)]
                new.append(a)
            return tuple(new)

        accs0 = (zero,) * (_UNROLL // 2)
        pre_accs = lax.fori_loop(0, base_vec // _UNROLL, count_body, accs0)
        tot_accs = lax.fori_loop(base_vec // _UNROLL, n_vecs_total // _UNROLL,
                                 count_body, pre_accs)
        ones_before = jnp.sum(sum(pre_accs, zero))
        ones_total = jnp.sum(sum(tot_accs, zero))
        zeros_total = n_rows - ones_total

        iota = lax.iota(jnp.int32, _L)

        def dest_body(k, ones_run):
            for j in range(vecs_per_chunk):
                kv = k * vecs_per_chunk + j
                v = pv[pl.ds((base_vec + kv) * _L, _L)]
                incl = plsc.cumsum(v)
                ones_excl = ones_run + incl - v
                row = base + kv * _L + iota
                dest = jnp.where(v == 0, row - ones_excl,
                                 zeros_total + ones_excl)
                idx[k, pl.ds(j * _L, _L)] = dest
                ones_run = ones_run + jnp.max(incl)
            return ones_run

        lax.fori_loop(0, n_chunks, dest_body, ones_before)

        def src_at(k):
            if _INDIRECT:
                return data_hbm.at[idx.at[k]]
            return data_hbm.at[pl.ds(base + k * _CHUNK, _CHUNK)]

        def dst_at(k):
            if _INDIRECT:
                return out_hbm.at[idx.at[k]]
            return out_hbm.at[pl.ds(base + k * _CHUNK, _CHUNK)]

        def start_in(k):
            pltpu.async_copy(src_at(k), buf.at[k % _NBUF], sin[k % _NBUF])

        def wait_in(k):
            pltpu.make_async_copy(src_at(k), buf.at[k % _NBUF],
                                  sin[k % _NBUF]).wait()

        def start_out(k):
            pltpu.async_copy(buf.at[k % _NBUF], dst_at(k), sout[k % _NBUF])

        def wait_out(k):
            pltpu.make_async_copy(buf.at[k % _NBUF], dst_at(k),
                                  sout[k % _NBUF]).wait()

        for k in range(n_chunks + _LAG):
            if k < n_chunks:
                if k >= _NBUF:
                    wait_out(k - _NBUF)
                start_in(k)
            j = k - _LAG
            if 0 <= j < n_chunks:
                wait_in(j)
                start_out(j)
        for j in range(n_chunks - _NBUF, n_chunks):
            wait_out(j)

    return run(data, partitions)
